# baseline (speedup 1.0000x reference)
import jax
import jax.numpy as jnp
from jax import lax
from jax.experimental import pallas as pl
from jax.experimental.pallas import tpu as pltpu
from jax.experimental.pallas import tpu_sc as plsc

N = 10000
NP = 10240
E = 320000
EP = 327680
W = 128
NWIN = EP // (32 * W)
WA = 256
NWA = EP // (32 * WA)
ROWS_PER_TILE = NP // 16
KB = 8

_mesh = plsc.VectorSubcoreMesh(
    core_axis_name="c", subcore_axis_name="s", num_cores=2, num_subcores=16)
_sc_params = pltpu.CompilerParams(
    use_tc_tiling_on_sc=False, disable_bounds_checks=True)
_sc_params_nolayout = pltpu.CompilerParams(
    use_tc_tiling_on_sc=False, needs_layout_passes=False,
    disable_bounds_checks=True)


def _deg_body(dst_hbm, zeros_hbm, out_hbm, idx_d, ones_v, acc, sems):
    c = lax.axis_index("c")
    s = lax.axis_index("s")
    wid = c * 16 + s
    for i in range(WA // 16):
        ones_v[pl.ds(i * 16, 16)] = jnp.full((16,), 1.0, jnp.float32)

    @pl.when(s == 0)
    def _():
        pltpu.sync_copy(zeros_hbm, acc)

    pltpu.sync_copy(dst_hbm.at[wid], idx_d)
    plsc.subcore_barrier()

    def body(i, _):
        descs = []
        for b in range(KB):
            w = i * KB + b
            descs.append(pltpu.async_copy(
                ones_v, acc.at[idx_d.at[w]], sems[b], add=True))
        for d in descs:
            d.wait()
        return _

    lax.fori_loop(0, NWA // KB, body, None)
    plsc.subcore_barrier()
    pltpu.sync_copy(acc.at[pl.ds(s * ROWS_PER_TILE, ROWS_PER_TILE)],
                    out_hbm.at[c, pl.ds(s * ROWS_PER_TILE, ROWS_PER_TILE)])


def _sc_degree(dst_pad, zeros_1d):
    return pl.kernel(
        _deg_body,
        out_type=jax.ShapeDtypeStruct((2, NP), jnp.float32),
        mesh=_mesh,
        compiler_params=_sc_params,
        scratch_types=[
            pltpu.VMEM((NWA, WA), jnp.int32),
            pltpu.VMEM((WA,), jnp.float32),
            pltpu.VMEM_SHARED((NP,), jnp.float32),
            [pltpu.SemaphoreType.DMA] * KB,
        ],
    )(dst_pad, zeros_1d)


def _sc_aggregate(ht, src_pad, dst_pad, zeros_2d):
    F = ht.shape[1]

    def body(ht_hbm, src_hbm, dst_hbm, zeros_hbm, out_hbm,
             idx_s, idx_d, rows, acc, semg, sems):
        c = lax.axis_index("c")
        s = lax.axis_index("s")
        wid = c * 16 + s

        @pl.when(s == 0)
        def _():
            pltpu.sync_copy(zeros_hbm, acc)

        pltpu.sync_copy(src_hbm.at[wid], idx_s)
        pltpu.sync_copy(dst_hbm.at[wid], idx_d)
        plsc.subcore_barrier()

        def loop(i, _):
            gd = []
            for b in range(KB):
                w = i * KB + b
                gd.append(pltpu.async_copy(
                    ht_hbm.at[idx_s.at[w]], rows[b], semg[b]))
            sd = []
            for b in range(KB):
                w = i * KB + b
                gd[b].wait()
                sd.append(pltpu.async_copy(
                    rows[b], acc.at[idx_d.at[w]], sems[b], add=True))
            for d in sd:
                d.wait()
            return _

        lax.fori_loop(0, NWA // KB, loop, None)
        plsc.subcore_barrier()
        pltpu.sync_copy(acc.at[pl.ds(s * ROWS_PER_TILE, ROWS_PER_TILE)],
                        out_hbm.at[c, pl.ds(s * ROWS_PER_TILE, ROWS_PER_TILE)])

    return pl.kernel(
        body,
        out_type=jax.ShapeDtypeStruct((2, NP, F), jnp.float32),
        mesh=_mesh,
        compiler_params=_sc_params,
        scratch_types=[
            pltpu.VMEM((NWA, WA), jnp.int32),
            pltpu.VMEM((NWA, WA), jnp.int32),
            [pltpu.VMEM((WA, F), jnp.float32)] * KB,
            pltpu.VMEM_SHARED((NP, F), jnp.float32),
            [pltpu.SemaphoreType.DMA] * KB,
            [pltpu.SemaphoreType.DMA] * KB,
        ],
    )(ht, src_pad, dst_pad, zeros_2d)


def _decode_window(rows_a, rows_b, out_v):
    iota = lax.iota(jnp.int32, 16)
    cids = [jnp.full((16,), j, jnp.int32) for j in range(16)]
    for g in range(W // 16):
        rid = iota + g * 16
        p = [jnp.zeros((16,), jnp.float32) for _ in range(8)]
        for j in range(16):
            va = plsc.load_gather(rows_a, [rid, cids[j]])
            vb = plsc.load_gather(rows_b, [rid, cids[j]])
            p[j % 8] = p[j % 8] + va * vb
        s0 = (p[0] + p[1]) + (p[2] + p[3])
        s1 = (p[4] + p[5]) + (p[6] + p[7])
        out_v[pl.ds(g * 16, 16)] = s0 + s1


KD = 8


def _decode_body(z2_hbm, sa_hbm, sb_hbm, out_hbm,
                 idx_a, idx_b, rows_a, rows_b, out_v, sema, semb, semo):
    c = lax.axis_index("c")
    s = lax.axis_index("s")
    wid = c * 16 + s
    chunk = wid * (NWIN * W)
    pltpu.sync_copy(sa_hbm.at[wid], idx_a)
    pltpu.sync_copy(sb_hbm.at[wid], idx_b)

    for b in range(KD):
        pltpu.async_copy(z2_hbm.at[idx_a.at[b]], rows_a[b], sema[b])
        pltpu.async_copy(z2_hbm.at[idx_b.at[b]], rows_b[b], semb[b])

    def loop(i, _):
        for b in range(KD):
            w = i * KD + b
            pltpu.make_async_copy(z2_hbm.at[idx_a.at[w]],
                                  rows_a[b], sema[b]).wait()
            pltpu.make_async_copy(z2_hbm.at[idx_b.at[w]],
                                  rows_b[b], semb[b]).wait()
            @pl.when(i > 0)
            def _():
                pltpu.make_async_copy(
                    out_v[b], out_hbm.at[pl.ds(chunk + (w - KD) * W, W)],
                    semo[b]).wait()
            _decode_window(rows_a[b], rows_b[b], out_v[b])
            pltpu.async_copy(
                out_v[b], out_hbm.at[pl.ds(chunk + w * W, W)], semo[b])

            @pl.when(w + KD < NWIN)
            def _():
                pltpu.async_copy(z2_hbm.at[idx_a.at[w + KD]],
                                 rows_a[b], sema[b])
                pltpu.async_copy(z2_hbm.at[idx_b.at[w + KD]],
                                 rows_b[b], semb[b])
        return _

    lax.fori_loop(0, NWIN // KD, loop, None)
    for b in range(KD):
        w = NWIN - KD + b
        pltpu.make_async_copy(
            out_v[b], out_hbm.at[pl.ds(chunk + w * W, W)], semo[b]).wait()


def _sc_decode(z2, sa_pad, sb_pad):
    return pl.kernel(
        _decode_body,
        out_type=jax.ShapeDtypeStruct((EP,), jnp.float32),
        mesh=_mesh,
        compiler_params=_sc_params_nolayout,
        scratch_types=[
            pltpu.VMEM((NWIN, W), jnp.int32),
            pltpu.VMEM((NWIN, W), jnp.int32),
            [pltpu.VMEM((W, 16), jnp.float32)] * KD,
            [pltpu.VMEM((W, 16), jnp.float32)] * KD,
            [pltpu.VMEM((W,), jnp.float32)] * KD,
            [pltpu.SemaphoreType.DMA] * KD,
            [pltpu.SemaphoreType.DMA] * KD,
            [pltpu.SemaphoreType.DMA] * KD,
        ],
    )(z2, sa_pad, sb_pad)


def _tc1_body(dp_ref, x_ref, w1_ref, dis_ref, h1t_ref):
    deg = dp_ref[0] + dp_ref[1] + 1.0
    dis = lax.rsqrt(deg)
    dis_ref[...] = dis
    h1t_ref[...] = jnp.dot(x_ref[...], w1_ref[...],
                           preferred_element_type=jnp.float32) * dis


def _tc1(deg_parts, x_pad, W1):
    return pl.pallas_call(
        _tc1_body,
        out_shape=(jax.ShapeDtypeStruct((NP, 1), jnp.float32),
                   jax.ShapeDtypeStruct((NP, 32), jnp.float32)),
    )(deg_parts, x_pad, W1)


def _tc2_body(a_ref, h1t_ref, dis_ref, b1_ref, w2_ref, h2t_ref):
    z1 = (a_ref[0] + a_ref[1] + h1t_ref[...]) * dis_ref[...] + b1_ref[...]
    z1 = jnp.maximum(z1, 0.0)
    h2t_ref[...] = jnp.dot(z1, w2_ref[...],
                           preferred_element_type=jnp.float32) * dis_ref[...]


def _tc2(acc1, h1t, dis, b1, W2):
    return pl.pallas_call(
        _tc2_body,
        out_shape=jax.ShapeDtypeStruct((NP, 16), jnp.float32),
    )(acc1, h1t, dis, b1, W2)


def _tc3_body(a_ref, h2t_ref, dis_ref, b2_ref, z2_ref):
    z2_ref[...] = ((a_ref[0] + a_ref[1] + h2t_ref[...]) * dis_ref[...]
                   + b2_ref[...])


def _tc3(acc2, h2t, dis, b2):
    return pl.pallas_call(
        _tc3_body,
        out_shape=jax.ShapeDtypeStruct((NP, 16), jnp.float32),
    )(acc2, h2t, dis, b2)


def kernel(x, edge_index, edge_label_index, W1, b1, W2, b2):
    pad_e = EP - E
    pad_ids = (jnp.arange(pad_e, dtype=jnp.int32) % (NP - N)) + N
    src_pad = jnp.concatenate([edge_index[0], pad_ids]).reshape(32, NWA, WA)
    dst_pad = jnp.concatenate([edge_index[1], pad_ids]).reshape(32, NWA, WA)
    sa_pad = jnp.concatenate([edge_label_index[0], pad_ids]).reshape(
        32, NWIN, W)
    sb_pad = jnp.concatenate([edge_label_index[1], pad_ids]).reshape(
        32, NWIN, W)
    x_pad = jnp.pad(x, ((0, NP - N), (0, 0)))
    zeros_1d = jnp.zeros((NP,), jnp.float32)
    zeros_32 = jnp.zeros((NP, 32), jnp.float32)
    zeros_16 = jnp.zeros((NP, 16), jnp.float32)

    deg_parts = _sc_degree(dst_pad, zeros_1d)
    dis, h1t = _tc1(deg_parts.reshape(2, NP, 1), x_pad, W1)
    acc1 = _sc_aggregate(h1t, src_pad, dst_pad, zeros_32)
    h2t = _tc2(acc1, h1t, dis, b1.reshape(1, 32), W2)
    acc2 = _sc_aggregate(h2t, src_pad, dst_pad, zeros_16)
    z2 = _tc3(acc2, h2t, dis, b2.reshape(1, 16))
    out = _sc_decode(z2, sa_pad, sb_pad)
    return out[:E]


# speedup vs baseline: 1.0501x; 1.0501x over previous
import jax
import jax.numpy as jnp
from jax import lax
from jax.experimental import pallas as pl
from jax.experimental.pallas import tpu as pltpu
from jax.experimental.pallas import tpu_sc as plsc

N = 10000
NP = 10240
E = 320000
EP = 327680
W = 128
NWIN = EP // (32 * W)
WA = 256
NWA = EP // (32 * WA)
ROWS_PER_TILE = NP // 16
KB = 8

_mesh = plsc.VectorSubcoreMesh(
    core_axis_name="c", subcore_axis_name="s", num_cores=2, num_subcores=16)
_sc_params = pltpu.CompilerParams(
    use_tc_tiling_on_sc=False, disable_bounds_checks=True)
_sc_params_nolayout = pltpu.CompilerParams(
    use_tc_tiling_on_sc=False, needs_layout_passes=False,
    disable_bounds_checks=True)


def _deg_body(dst_hbm, zeros_hbm, out_hbm, idx_d, ones_v, acc, sems):
    c = lax.axis_index("c")
    s = lax.axis_index("s")
    wid = c * 16 + s
    for i in range(WA // 16):
        ones_v[pl.ds(i * 16, 16)] = jnp.full((16,), 1.0, jnp.float32)

    @pl.when(s == 0)
    def _():
        pltpu.sync_copy(zeros_hbm, acc)

    pltpu.sync_copy(dst_hbm.at[wid], idx_d)
    plsc.subcore_barrier()

    def body(i, _):
        descs = []
        for b in range(KB):
            w = i * KB + b
            descs.append(pltpu.async_copy(
                ones_v, acc.at[idx_d.at[w]], sems[b], add=True))
        for d in descs:
            d.wait()
        return _

    lax.fori_loop(0, NWA // KB, body, None)
    plsc.subcore_barrier()
    pltpu.sync_copy(acc.at[pl.ds(s * ROWS_PER_TILE, ROWS_PER_TILE)],
                    out_hbm.at[c, pl.ds(s * ROWS_PER_TILE, ROWS_PER_TILE)])


def _sc_degree(dst_pad, zeros_1d):
    return pl.kernel(
        _deg_body,
        out_type=jax.ShapeDtypeStruct((2, NP), jnp.float32),
        mesh=_mesh,
        compiler_params=_sc_params,
        scratch_types=[
            pltpu.VMEM((NWA, WA), jnp.int32),
            pltpu.VMEM((WA,), jnp.float32),
            pltpu.VMEM_SHARED((NP,), jnp.float32),
            [pltpu.SemaphoreType.DMA] * KB,
        ],
    )(dst_pad, zeros_1d)


def _sc_aggregate(ht, src_pad, dst_pad, zeros_2d):
    F = ht.shape[1]

    def body(ht_hbm, src_hbm, dst_hbm, zeros_hbm, out_hbm,
             idx_s, idx_d, rows, acc, semg, sems):
        c = lax.axis_index("c")
        s = lax.axis_index("s")
        wid = c * 16 + s

        @pl.when(s == 0)
        def _():
            pltpu.sync_copy(zeros_hbm, acc)

        pltpu.sync_copy(src_hbm.at[wid], idx_s)
        pltpu.sync_copy(dst_hbm.at[wid], idx_d)
        plsc.subcore_barrier()

        for b in range(KB):
            pltpu.async_copy(ht_hbm.at[idx_s.at[b]], rows[b], semg[b])

        def loop(i, _):
            for b in range(KB):
                w = i * KB + b
                pltpu.make_async_copy(ht_hbm.at[idx_s.at[w]],
                                      rows[b], semg[b]).wait()
                pltpu.async_copy(rows[b], acc.at[idx_d.at[w]],
                                 sems[b], add=True)
                pltpu.make_async_copy(rows[b], acc.at[idx_d.at[w]],
                                      sems[b]).wait()

                @pl.when(w + KB < NWA)
                def _():
                    pltpu.async_copy(ht_hbm.at[idx_s.at[w + KB]],
                                     rows[b], semg[b])
            return _

        lax.fori_loop(0, NWA // KB, loop, None)
        plsc.subcore_barrier()
        pltpu.sync_copy(acc.at[pl.ds(s * ROWS_PER_TILE, ROWS_PER_TILE)],
                        out_hbm.at[c, pl.ds(s * ROWS_PER_TILE, ROWS_PER_TILE)])

    return pl.kernel(
        body,
        out_type=jax.ShapeDtypeStruct((2, NP, F), jnp.float32),
        mesh=_mesh,
        compiler_params=_sc_params,
        scratch_types=[
            pltpu.VMEM((NWA, WA), jnp.int32),
            pltpu.VMEM((NWA, WA), jnp.int32),
            [pltpu.VMEM((WA, F), jnp.float32)] * KB,
            pltpu.VMEM_SHARED((NP, F), jnp.float32),
            [pltpu.SemaphoreType.DMA] * KB,
            [pltpu.SemaphoreType.DMA] * KB,
        ],
    )(ht, src_pad, dst_pad, zeros_2d)


def _decode_window(rows_a, rows_b, out_v):
    iota = lax.iota(jnp.int32, 16)
    cids = [jnp.full((16,), j, jnp.int32) for j in range(16)]
    for g in range(W // 16):
        rid = iota + g * 16
        p = [jnp.zeros((16,), jnp.float32) for _ in range(8)]
        for j in range(16):
            va = plsc.load_gather(rows_a, [rid, cids[j]])
            vb = plsc.load_gather(rows_b, [rid, cids[j]])
            p[j % 8] = p[j % 8] + va * vb
        s0 = (p[0] + p[1]) + (p[2] + p[3])
        s1 = (p[4] + p[5]) + (p[6] + p[7])
        out_v[pl.ds(g * 16, 16)] = s0 + s1


KD = 4


def _decode_body(z2_hbm, sa_hbm, sb_hbm, out_hbm,
                 idx_a, idx_b, rows_a, rows_b, out_v, sema, semb, semo):
    c = lax.axis_index("c")
    s = lax.axis_index("s")
    wid = c * 16 + s
    chunk = wid * (NWIN * W)
    pltpu.sync_copy(sa_hbm.at[wid], idx_a)
    pltpu.sync_copy(sb_hbm.at[wid], idx_b)

    for b in range(KD):
        pltpu.async_copy(z2_hbm.at[idx_a.at[b]], rows_a[b], sema[b])
        pltpu.async_copy(z2_hbm.at[idx_b.at[b]], rows_b[b], semb[b])

    def loop(i, _):
        for b in range(KD):
            w = i * KD + b
            pltpu.make_async_copy(z2_hbm.at[idx_a.at[w]],
                                  rows_a[b], sema[b]).wait()
            pltpu.make_async_copy(z2_hbm.at[idx_b.at[w]],
                                  rows_b[b], semb[b]).wait()
            @pl.when(i > 0)
            def _():
                pltpu.make_async_copy(
                    out_v[b], out_hbm.at[pl.ds(chunk + (w - KD) * W, W)],
                    semo[b]).wait()
            _decode_window(rows_a[b], rows_b[b], out_v[b])
            pltpu.async_copy(
                out_v[b], out_hbm.at[pl.ds(chunk + w * W, W)], semo[b])

            @pl.when(w + KD < NWIN)
            def _():
                pltpu.async_copy(z2_hbm.at[idx_a.at[w + KD]],
                                 rows_a[b], sema[b])
                pltpu.async_copy(z2_hbm.at[idx_b.at[w + KD]],
                                 rows_b[b], semb[b])
        return _

    lax.fori_loop(0, NWIN // KD, loop, None)
    for b in range(KD):
        w = NWIN - KD + b
        pltpu.make_async_copy(
            out_v[b], out_hbm.at[pl.ds(chunk + w * W, W)], semo[b]).wait()


def _sc_decode(z2, sa_pad, sb_pad):
    return pl.kernel(
        _decode_body,
        out_type=jax.ShapeDtypeStruct((EP,), jnp.float32),
        mesh=_mesh,
        compiler_params=_sc_params_nolayout,
        scratch_types=[
            pltpu.VMEM((NWIN, W), jnp.int32),
            pltpu.VMEM((NWIN, W), jnp.int32),
            [pltpu.VMEM((W, 16), jnp.float32)] * KD,
            [pltpu.VMEM((W, 16), jnp.float32)] * KD,
            [pltpu.VMEM((W,), jnp.float32)] * KD,
            [pltpu.SemaphoreType.DMA] * KD,
            [pltpu.SemaphoreType.DMA] * KD,
            [pltpu.SemaphoreType.DMA] * KD,
        ],
    )(z2, sa_pad, sb_pad)


def _tc1_body(dp_ref, x_ref, w1_ref, dis_ref, h1t_ref):
    deg = dp_ref[0] + dp_ref[1] + 1.0
    dis = lax.rsqrt(deg)
    dis_ref[...] = dis
    h1t_ref[...] = jnp.dot(x_ref[...], w1_ref[...],
                           preferred_element_type=jnp.float32) * dis


def _tc1(deg_parts, x_pad, W1):
    return pl.pallas_call(
        _tc1_body,
        out_shape=(jax.ShapeDtypeStruct((NP, 1), jnp.float32),
                   jax.ShapeDtypeStruct((NP, 32), jnp.float32)),
    )(deg_parts, x_pad, W1)


def _tc2_body(a_ref, h1t_ref, dis_ref, b1_ref, w2_ref, h2t_ref):
    z1 = (a_ref[0] + a_ref[1] + h1t_ref[...]) * dis_ref[...] + b1_ref[...]
    z1 = jnp.maximum(z1, 0.0)
    h2t_ref[...] = jnp.dot(z1, w2_ref[...],
                           preferred_element_type=jnp.float32) * dis_ref[...]


def _tc2(acc1, h1t, dis, b1, W2):
    return pl.pallas_call(
        _tc2_body,
        out_shape=jax.ShapeDtypeStruct((NP, 16), jnp.float32),
    )(acc1, h1t, dis, b1, W2)


def _tc3_body(a_ref, h2t_ref, dis_ref, b2_ref, z2_ref):
    z2_ref[...] = ((a_ref[0] + a_ref[1] + h2t_ref[...]) * dis_ref[...]
                   + b2_ref[...])


def _tc3(acc2, h2t, dis, b2):
    return pl.pallas_call(
        _tc3_body,
        out_shape=jax.ShapeDtypeStruct((NP, 16), jnp.float32),
    )(acc2, h2t, dis, b2)


def kernel(x, edge_index, edge_label_index, W1, b1, W2, b2):
    pad_e = EP - E
    pad_ids = (jnp.arange(pad_e, dtype=jnp.int32) % (NP - N)) + N
    src_pad = jnp.concatenate([edge_index[0], pad_ids]).reshape(32, NWA, WA)
    dst_pad = jnp.concatenate([edge_index[1], pad_ids]).reshape(32, NWA, WA)
    sa_pad = jnp.concatenate([edge_label_index[0], pad_ids]).reshape(
        32, NWIN, W)
    sb_pad = jnp.concatenate([edge_label_index[1], pad_ids]).reshape(
        32, NWIN, W)
    x_pad = jnp.pad(x, ((0, NP - N), (0, 0)))
    zeros_1d = jnp.zeros((NP,), jnp.float32)
    zeros_32 = jnp.zeros((NP, 32), jnp.float32)
    zeros_16 = jnp.zeros((NP, 16), jnp.float32)

    deg_parts = _sc_degree(dst_pad, zeros_1d)
    dis, h1t = _tc1(deg_parts.reshape(2, NP, 1), x_pad, W1)
    acc1 = _sc_aggregate(h1t, src_pad, dst_pad, zeros_32)
    h2t = _tc2(acc1, h1t, dis, b1.reshape(1, 32), W2)
    acc2 = _sc_aggregate(h2t, src_pad, dst_pad, zeros_16)
    z2 = _tc3(acc2, h2t, dis, b2.reshape(1, 16))
    out = _sc_decode(z2, sa_pad, sb_pad)
    return out[:E]
